# baseline (speedup 1.0000x reference)
import functools

import jax
import jax.numpy as jnp
from jax import lax
from jax.experimental import pallas as pl
from jax.experimental.pallas import tpu as pltpu
from jax.experimental.pallas import tpu_sc as plsc

N = 10000
K = 32
C = 128
MID = 128
S = 8
CS = C // S
EPS = 1e-5
E = N * K

P_BLK = 200
E_BLK = P_BLK * K
GRID = N // P_BLK
N_BLK = 2000

_pcall = pl.pallas_call


def _proj_body(f_ref, wq_ref, bq_ref, wk_ref, bk_ref, wv_ref, bv_ref,
               q_ref, k_ref, v_ref):
    f = f_ref[...]
    q_ref[...] = jnp.dot(f, wq_ref[...]) + bq_ref[...]
    k_ref[...] = jnp.dot(f, wk_ref[...]) + bk_ref[...]
    v_ref[...] = jnp.dot(f, wv_ref[...]) + bv_ref[...]


def _proj(feats, Wq, bq, Wk, bk, Wv, bv):
    row = pl.BlockSpec((N_BLK, C), lambda i: (i, 0))
    full = pl.BlockSpec((C, C), lambda i: (0, 0))
    bias = pl.BlockSpec((1, C), lambda i: (0, 0))
    return _pcall(
        _proj_body,
        grid=(N // N_BLK,),
        in_specs=[row, full, bias, full, bias, full, bias],
        out_specs=[row, row, row],
        out_shape=[jax.ShapeDtypeStruct((N, C), jnp.float32)] * 3,
    )(feats, Wq, bq.reshape(1, C), Wk, bk.reshape(1, C),
      Wv, bv.reshape(1, C))


def _sc_gather_one(tbl, idx, width, nsplit=1):
    info = plsc.get_sparse_core_info()
    nc, ns = info.num_cores, info.num_subcores
    nw = nc * ns
    b_per_w = E // nw
    ch = 80
    nb = 5
    chs = ch // nsplit
    n_it = b_per_w // (ch * nb)
    mesh = plsc.VectorSubcoreMesh(core_axis_name="c", subcore_axis_name="s")

    @functools.partial(
        pl.kernel, mesh=mesh,
        compiler_params=pltpu.CompilerParams(use_tc_tiling_on_sc=False),
        out_type=jax.ShapeDtypeStruct((E, width), jnp.float32),
        scratch_types=[
            pltpu.VMEM((b_per_w,), jnp.int32),
            pltpu.VMEM((nb, ch, width), jnp.float32),
        ] + [pltpu.SemaphoreType.DMA] * (2 * nb),
    )
    def gather(tbl_hbm, idx_hbm, out_hbm, idx_v, buf_v, *sems):
        gsem, wsem = sems[:nb], sems[nb:]
        wid = lax.axis_index("s") * nc + lax.axis_index("c")
        base = wid * b_per_w
        pltpu.sync_copy(idx_hbm.at[pl.ds(base, b_per_w)], idx_v)

        def drain_write(b):
            pltpu.make_async_copy(buf_v.at[b], out_hbm.at[pl.ds(base, ch)],
                                  wsem[b]).wait()

        def body(it, carry):
            descs = []
            for b in range(nb):
                c = it * nb + b

                @pl.when(it > 0)
                def _(b=b):
                    drain_write(b)

                descs.append([
                    pltpu.async_copy(
                        tbl_hbm.at[idx_v.at[pl.ds(c * ch + s * chs, chs)]],
                        buf_v.at[b, pl.ds(s * chs, chs)], gsem[b])
                    for s in range(nsplit)])
            for b in range(nb):
                off = base + (it * nb + b) * ch
                for dsc in descs[b]:
                    dsc.wait()
                pltpu.async_copy(buf_v.at[b], out_hbm.at[pl.ds(off, ch)],
                                 wsem[b])
            return carry

        lax.fori_loop(0, n_it, body, 0)
        for b in range(nb):
            drain_write(b)

    return gather(tbl, idx)


def _edge_d(gx, xp, p1w, p1b, s1a, s1b, p2w, p2b):
    gx3 = gx.reshape(P_BLK, K, 16)
    dx = (gx3 - xp[:, None, :]).reshape(E_BLK, 16)
    x = jnp.dot(dx, p1w) + p1b
    r = jnp.maximum(x * s1a + s1b, 0.0)
    return jnp.dot(r, p2w) + p2b


def _acc_update(acc, i, vals, out_ref):
    @pl.when(i == 0)
    def _():
        acc[...] = jnp.zeros_like(acc[...])

    acc[0] += vals.sum(axis=0)
    acc[1] += (vals * vals).sum(axis=0)

    @pl.when(i == pl.num_programs(0) - 1)
    def _():
        out_ref[...] = acc[...]


def _stats1_body(gx_ref, xp_ref, p1w_ref, p1b_ref, out_ref, acc):
    i = pl.program_id(0)
    gx3 = gx_ref[...].reshape(P_BLK, K, 16)
    dx = (gx3 - xp_ref[...][:, None, :]).reshape(E_BLK, 16)
    x = jnp.dot(dx, p1w_ref[...]) + p1b_ref[...]
    _acc_update(acc, i, x.reshape(E_BLK // 8, 8, 16), out_ref)


def _stats1(gx, xp, p1w, p1b):
    return _pcall(
        _stats1_body,
        grid=(GRID,),
        in_specs=[
            pl.BlockSpec((E_BLK, 16), lambda i: (i, 0)),
            pl.BlockSpec((P_BLK, 16), lambda i: (i, 0)),
            pl.BlockSpec((16, 16), lambda i: (0, 0)),
            pl.BlockSpec((1, 16), lambda i: (0, 0)),
        ],
        out_specs=pl.BlockSpec((2, 8, 16), lambda i: (0, 0, 0)),
        out_shape=jax.ShapeDtypeStruct((2, 8, 16), jnp.float32),
        scratch_shapes=[pltpu.VMEM((2, 8, 16), jnp.float32)],
    )(gx, xp, p1w, p1b)


def _stats2_body(gk_ref, gx_ref, q_ref, xp_ref, s1a_ref, s1b_ref,
                 p1w_ref, p1b_ref, p2w_ref, p2b_ref, out_ref, acc):
    i = pl.program_id(0)
    d = _edge_d(gx_ref[...], xp_ref[...], p1w_ref[...], p1b_ref[...],
                s1a_ref[...], s1b_ref[...], p2w_ref[...], p2b_ref[...])
    w3 = (gk_ref[...].reshape(P_BLK, K, C)
          - q_ref[...][:, None, :] + d.reshape(P_BLK, K, C))
    _acc_update(acc, i, w3.reshape(E_BLK // 8, 8, C), out_ref)


def _stats2(gk, gx, q, xp, s1a, s1b, p1w, p1b, p2w, p2b):
    return _pcall(
        _stats2_body,
        grid=(GRID,),
        in_specs=[
            pl.BlockSpec((E_BLK, C), lambda i: (i, 0)),
            pl.BlockSpec((E_BLK, 16), lambda i: (i, 0)),
            pl.BlockSpec((P_BLK, C), lambda i: (i, 0)),
            pl.BlockSpec((P_BLK, 16), lambda i: (i, 0)),
            pl.BlockSpec((1, 16), lambda i: (0, 0)),
            pl.BlockSpec((1, 16), lambda i: (0, 0)),
            pl.BlockSpec((16, 16), lambda i: (0, 0)),
            pl.BlockSpec((1, 16), lambda i: (0, 0)),
            pl.BlockSpec((16, C), lambda i: (0, 0)),
            pl.BlockSpec((1, C), lambda i: (0, 0)),
        ],
        out_specs=pl.BlockSpec((2, 8, C), lambda i: (0, 0, 0)),
        out_shape=jax.ShapeDtypeStruct((2, 8, C), jnp.float32),
        scratch_shapes=[pltpu.VMEM((2, 8, C), jnp.float32)],
    )(gk, gx, q, xp, s1a, s1b, p1w, p1b, p2w, p2b)


def _stats3_body(gk_ref, gx_ref, q_ref, xp_ref, s1a_ref, s1b_ref,
                 p1w_ref, p1b_ref, p2w_ref, p2b_ref, s2a_ref, s2b_ref,
                 w1w_ref, w1b_ref, h_ref, out_ref, acc):
    i = pl.program_id(0)
    d = _edge_d(gx_ref[...], xp_ref[...], p1w_ref[...], p1b_ref[...],
                s1a_ref[...], s1b_ref[...], p2w_ref[...], p2b_ref[...])
    w3 = (gk_ref[...].reshape(P_BLK, K, C)
          - q_ref[...][:, None, :] + d.reshape(P_BLK, K, C))
    wn = jnp.maximum(w3.reshape(E_BLK, C) * s2a_ref[...] + s2b_ref[...], 0.0)
    h = jnp.dot(wn, w1w_ref[...]) + w1b_ref[...]
    h_ref[...] = h
    _acc_update(acc, i, h.reshape(E_BLK // 8, 8, CS), out_ref)


def _stats3(gk, gx, q, xp, s1a, s1b, p1w, p1b, p2w, p2b, s2a, s2b, w1w, w1b):
    return _pcall(
        _stats3_body,
        grid=(GRID,),
        in_specs=[
            pl.BlockSpec((E_BLK, C), lambda i: (i, 0)),
            pl.BlockSpec((E_BLK, 16), lambda i: (i, 0)),
            pl.BlockSpec((P_BLK, C), lambda i: (i, 0)),
            pl.BlockSpec((P_BLK, 16), lambda i: (i, 0)),
            pl.BlockSpec((1, 16), lambda i: (0, 0)),
            pl.BlockSpec((1, 16), lambda i: (0, 0)),
            pl.BlockSpec((16, 16), lambda i: (0, 0)),
            pl.BlockSpec((1, 16), lambda i: (0, 0)),
            pl.BlockSpec((16, C), lambda i: (0, 0)),
            pl.BlockSpec((1, C), lambda i: (0, 0)),
            pl.BlockSpec((1, C), lambda i: (0, 0)),
            pl.BlockSpec((1, C), lambda i: (0, 0)),
            pl.BlockSpec((C, CS), lambda i: (0, 0)),
            pl.BlockSpec((1, CS), lambda i: (0, 0)),
        ],
        out_specs=[
            pl.BlockSpec((E_BLK, CS), lambda i: (i, 0)),
            pl.BlockSpec((2, 8, CS), lambda i: (0, 0, 0)),
        ],
        out_shape=[
            jax.ShapeDtypeStruct((E, CS), jnp.float32),
            jax.ShapeDtypeStruct((2, 8, CS), jnp.float32),
        ],
        scratch_shapes=[pltpu.VMEM((2, 8, CS), jnp.float32)],
    )(gk, gx, q, xp, s1a, s1b, p1w, p1b, p2w, p2b, s2a, s2b, w1w, w1b)


def _final_body(h_ref, gv_ref, gx_ref, xp_ref, f_ref, s1a_ref, s1b_ref,
                p1w_ref, p1b_ref, p2w_ref, p2b_ref, s3a_ref, s3b_ref,
                w2t_ref, w2bt_ref, out_ref):
    d = _edge_d(gx_ref[...], xp_ref[...], p1w_ref[...], p1b_ref[...],
                s1a_ref[...], s1b_ref[...], p2w_ref[...], p2b_ref[...])
    hn = jnp.maximum(h_ref[...] * s3a_ref[...] + s3b_ref[...], 0.0)
    gt = jnp.dot(hn, w2t_ref[...]) + w2bt_ref[...]
    m8 = gt.reshape(E_BLK // 8, 8, C).max(axis=1)
    m = m8.reshape(P_BLK, K // 8, C).max(axis=1)
    ex = jnp.exp(gt.reshape(P_BLK, K, C)
                 - m[:, None, :]).reshape(E_BLK, C)
    val = (gv_ref[...] + d) * ex
    n8 = val.reshape(E_BLK // 8, 8, C).sum(axis=1)
    num = n8.reshape(P_BLK, K // 8, C).sum(axis=1)
    d8 = ex.reshape(E_BLK // 8, 8, C).sum(axis=1)
    den = d8.reshape(P_BLK, K // 8, C).sum(axis=1)
    o = num / den + f_ref[...]
    out_ref[...] = jnp.where(o > 0, o, 0.1 * o)


def _final(h, gv, gx, xp, feats, s1a, s1b, p1w, p1b, p2w, p2b,
           s3a, s3b, w2t, w2bt):
    return _pcall(
        _final_body,
        grid=(GRID,),
        in_specs=[
            pl.BlockSpec((E_BLK, CS), lambda i: (i, 0)),
            pl.BlockSpec((E_BLK, C), lambda i: (i, 0)),
            pl.BlockSpec((E_BLK, 16), lambda i: (i, 0)),
            pl.BlockSpec((P_BLK, 16), lambda i: (i, 0)),
            pl.BlockSpec((P_BLK, C), lambda i: (i, 0)),
            pl.BlockSpec((1, 16), lambda i: (0, 0)),
            pl.BlockSpec((1, 16), lambda i: (0, 0)),
            pl.BlockSpec((16, 16), lambda i: (0, 0)),
            pl.BlockSpec((1, 16), lambda i: (0, 0)),
            pl.BlockSpec((16, C), lambda i: (0, 0)),
            pl.BlockSpec((1, C), lambda i: (0, 0)),
            pl.BlockSpec((1, CS), lambda i: (0, 0)),
            pl.BlockSpec((1, CS), lambda i: (0, 0)),
            pl.BlockSpec((CS, C), lambda i: (0, 0)),
            pl.BlockSpec((1, C), lambda i: (0, 0)),
        ],
        out_specs=pl.BlockSpec((P_BLK, C), lambda i: (i, 0)),
        out_shape=jax.ShapeDtypeStruct((N, C), jnp.float32),
    )(h, gv, gx, xp, feats, s1a, s1b, p1w, p1b, p2w, p2b,
      s3a, s3b, w2t, w2bt)


def _bn_fold(sums, gamma, beta):
    s = sums.sum(axis=1)
    m = s[0] / E
    v = s[1] / E - m * m
    a = gamma / jnp.sqrt(v + EPS)
    return (a.reshape(1, -1), (beta - m * a).reshape(1, -1))


def kernel(xyz, feats, nei_ind, Wq, bq, Wk, bk, Wv, bv, p1_W, p1_b,
           p1_gamma, p1_beta, p2_W, p2_b, bnw_gamma, bnw_beta, w1_W, w1_b,
           w1_gamma, w1_beta, w2_W, w2_b):
    f2 = feats[0]
    xp = jnp.pad(xyz[0], ((0, 0), (0, 13)))
    idx = nei_ind[0].reshape(E).astype(jnp.int32)

    p1w = jnp.zeros((16, 16), jnp.float32).at[:3, :3].set(p1_W)
    p1b = jnp.pad(p1_b, (0, 13)).reshape(1, 16)
    p1g = jnp.pad(p1_gamma, (0, 13))
    p1be = jnp.pad(p1_beta, (0, 13))
    p2w = jnp.zeros((16, C), jnp.float32).at[:3, :].set(p2_W)
    p2b = p2_b.reshape(1, C)

    gx = _sc_gather_one(xp, idx, 16)
    q, kf, vf = _proj(f2, Wq, bq, Wk, bk, Wv, bv)
    gk = _sc_gather_one(kf, idx, C, nsplit=2)
    gv = _sc_gather_one(vf, idx, C, nsplit=2)

    s1 = _stats1(gx, xp, p1w, p1b)
    s1a, s1b = _bn_fold(s1, p1g, p1be)

    s2 = _stats2(gk, gx, q, xp, s1a, s1b, p1w, p1b, p2w, p2b)
    s2a, s2b = _bn_fold(s2, bnw_gamma, bnw_beta)

    h, s3 = _stats3(gk, gx, q, xp, s1a, s1b, p1w, p1b, p2w, p2b,
                    s2a, s2b, w1_W, w1_b.reshape(1, CS))
    s3a, s3b = _bn_fold(s3, w1_gamma, w1_beta)

    out = _final(h, gv, gx, xp, f2, s1a, s1b, p1w, p1b, p2w, p2b,
                 s3a, s3b, jnp.tile(w2_W, (1, S)),
                 jnp.tile(w2_b, S).reshape(1, C))
    return out.reshape(1, N, C)


# speedup vs baseline: 1.0720x; 1.0720x over previous
import functools

import jax
import jax.numpy as jnp
from jax import lax
from jax.experimental import pallas as pl
from jax.experimental.pallas import tpu as pltpu
from jax.experimental.pallas import tpu_sc as plsc

N = 10000
K = 32
C = 128
MID = 128
S = 8
CS = C // S
EPS = 1e-5
E = N * K

P_BLK = 200
E_BLK = P_BLK * K
GRID = N // P_BLK
N_BLK = 2000

_pcall = pl.pallas_call


def _proj_body(f_ref, wq_ref, bq_ref, wk_ref, bk_ref, wv_ref, bv_ref,
               q_ref, k_ref, v_ref):
    f = f_ref[...]
    q_ref[...] = jnp.dot(f, wq_ref[...]) + bq_ref[...]
    k_ref[...] = jnp.dot(f, wk_ref[...]) + bk_ref[...]
    v_ref[...] = jnp.dot(f, wv_ref[...]) + bv_ref[...]


def _proj(feats, Wq, bq, Wk, bk, Wv, bv):
    row = pl.BlockSpec((N_BLK, C), lambda i: (i, 0))
    full = pl.BlockSpec((C, C), lambda i: (0, 0))
    bias = pl.BlockSpec((1, C), lambda i: (0, 0))
    return _pcall(
        _proj_body,
        grid=(N // N_BLK,),
        in_specs=[row, full, bias, full, bias, full, bias],
        out_specs=[row, row, row],
        out_shape=[jax.ShapeDtypeStruct((N, C), jnp.float32)] * 3,
    )(feats, Wq, bq.reshape(1, C), Wk, bk.reshape(1, C),
      Wv, bv.reshape(1, C))


def _sc_gather_one(tbl, idx, width, nsplit=1):
    info = plsc.get_sparse_core_info()
    nc, ns = info.num_cores, info.num_subcores
    nw = nc * ns
    b_per_w = E // nw
    ch = 80
    nb = 5
    chs = ch // nsplit
    n_it = b_per_w // (ch * nb)
    mesh = plsc.VectorSubcoreMesh(core_axis_name="c", subcore_axis_name="s")

    @functools.partial(
        pl.kernel, mesh=mesh,
        compiler_params=pltpu.CompilerParams(use_tc_tiling_on_sc=False),
        out_type=jax.ShapeDtypeStruct((E, width), jnp.float32),
        scratch_types=[
            pltpu.VMEM((b_per_w,), jnp.int32),
            pltpu.VMEM((nb, ch, width), jnp.float32),
        ] + [pltpu.SemaphoreType.DMA] * (2 * nb),
    )
    def gather(tbl_hbm, idx_hbm, out_hbm, idx_v, buf_v, *sems):
        gsem, wsem = sems[:nb], sems[nb:]
        wid = lax.axis_index("s") * nc + lax.axis_index("c")
        base = wid * b_per_w
        pltpu.sync_copy(idx_hbm.at[pl.ds(base, b_per_w)], idx_v)

        def drain_write(b):
            pltpu.make_async_copy(buf_v.at[b], out_hbm.at[pl.ds(base, ch)],
                                  wsem[b]).wait()

        def body(it, carry):
            descs = []
            for b in range(nb):
                c = it * nb + b

                @pl.when(it > 0)
                def _(b=b):
                    drain_write(b)

                descs.append([
                    pltpu.async_copy(
                        tbl_hbm.at[idx_v.at[pl.ds(c * ch + s * chs, chs)]],
                        buf_v.at[b, pl.ds(s * chs, chs)], gsem[b])
                    for s in range(nsplit)])
            for b in range(nb):
                off = base + (it * nb + b) * ch
                for dsc in descs[b]:
                    dsc.wait()
                pltpu.async_copy(buf_v.at[b], out_hbm.at[pl.ds(off, ch)],
                                 wsem[b])
            return carry

        lax.fori_loop(0, n_it, body, 0)
        for b in range(nb):
            drain_write(b)

    return gather(tbl, idx)


def _edge_d(gx, xp, p1w, p1b, s1a, s1b, p2w, p2b):
    gx3 = gx.reshape(P_BLK, K, 16)
    dx = (gx3 - xp[:, None, :]).reshape(E_BLK, 16)
    x = jnp.dot(dx, p1w) + p1b
    r = jnp.maximum(x * s1a + s1b, 0.0)
    return jnp.dot(r, p2w) + p2b


def _acc_update(acc, i, vals, out_ref):
    @pl.when(i == 0)
    def _():
        acc[...] = jnp.zeros_like(acc[...])

    acc[0] += vals.sum(axis=0)
    acc[1] += (vals * vals).sum(axis=0)

    @pl.when(i == pl.num_programs(0) - 1)
    def _():
        out_ref[...] = acc[...]


def _stats1_body(gx_ref, xp_ref, p1w_ref, p1b_ref, out_ref, acc):
    i = pl.program_id(0)
    gx3 = gx_ref[...].reshape(P_BLK, K, 16)
    dx = (gx3 - xp_ref[...][:, None, :]).reshape(E_BLK, 16)
    x = jnp.dot(dx, p1w_ref[...]) + p1b_ref[...]
    _acc_update(acc, i, x.reshape(E_BLK // 8, 8, 16), out_ref)


def _stats1(gx, xp, p1w, p1b):
    return _pcall(
        _stats1_body,
        grid=(GRID,),
        in_specs=[
            pl.BlockSpec((E_BLK, 16), lambda i: (i, 0)),
            pl.BlockSpec((P_BLK, 16), lambda i: (i, 0)),
            pl.BlockSpec((16, 16), lambda i: (0, 0)),
            pl.BlockSpec((1, 16), lambda i: (0, 0)),
        ],
        out_specs=pl.BlockSpec((2, 8, 16), lambda i: (0, 0, 0)),
        out_shape=jax.ShapeDtypeStruct((2, 8, 16), jnp.float32),
        scratch_shapes=[pltpu.VMEM((2, 8, 16), jnp.float32)],
    )(gx, xp, p1w, p1b)


def _stats2_body(gk_ref, gx_ref, q_ref, xp_ref, s1a_ref, s1b_ref,
                 p1w_ref, p1b_ref, p2w_ref, p2b_ref, out_ref, acc):
    i = pl.program_id(0)
    d = _edge_d(gx_ref[...], xp_ref[...], p1w_ref[...], p1b_ref[...],
                s1a_ref[...], s1b_ref[...], p2w_ref[...], p2b_ref[...])
    w3 = (gk_ref[...].reshape(P_BLK, K, C)
          - q_ref[...][:, None, :] + d.reshape(P_BLK, K, C))
    _acc_update(acc, i, w3.reshape(E_BLK // 8, 8, C), out_ref)


def _stats2(gk, gx, q, xp, s1a, s1b, p1w, p1b, p2w, p2b):
    return _pcall(
        _stats2_body,
        grid=(GRID,),
        in_specs=[
            pl.BlockSpec((E_BLK, C), lambda i: (i, 0)),
            pl.BlockSpec((E_BLK, 16), lambda i: (i, 0)),
            pl.BlockSpec((P_BLK, C), lambda i: (i, 0)),
            pl.BlockSpec((P_BLK, 16), lambda i: (i, 0)),
            pl.BlockSpec((1, 16), lambda i: (0, 0)),
            pl.BlockSpec((1, 16), lambda i: (0, 0)),
            pl.BlockSpec((16, 16), lambda i: (0, 0)),
            pl.BlockSpec((1, 16), lambda i: (0, 0)),
            pl.BlockSpec((16, C), lambda i: (0, 0)),
            pl.BlockSpec((1, C), lambda i: (0, 0)),
        ],
        out_specs=pl.BlockSpec((2, 8, C), lambda i: (0, 0, 0)),
        out_shape=jax.ShapeDtypeStruct((2, 8, C), jnp.float32),
        scratch_shapes=[pltpu.VMEM((2, 8, C), jnp.float32)],
    )(gk, gx, q, xp, s1a, s1b, p1w, p1b, p2w, p2b)


def _stats3_body(gk_ref, gx_ref, q_ref, xp_ref, s1a_ref, s1b_ref,
                 p1w_ref, p1b_ref, p2w_ref, p2b_ref, s2a_ref, s2b_ref,
                 w1w_ref, w1b_ref, h_ref, out_ref, acc):
    i = pl.program_id(0)
    d = _edge_d(gx_ref[...], xp_ref[...], p1w_ref[...], p1b_ref[...],
                s1a_ref[...], s1b_ref[...], p2w_ref[...], p2b_ref[...])
    w3 = (gk_ref[...].reshape(P_BLK, K, C)
          - q_ref[...][:, None, :] + d.reshape(P_BLK, K, C))
    wn = jnp.maximum(w3.reshape(E_BLK, C) * s2a_ref[...] + s2b_ref[...], 0.0)
    h = jnp.dot(wn, w1w_ref[...]) + w1b_ref[...]
    h_ref[...] = h
    _acc_update(acc, i, h.reshape(E_BLK // 8, 8, CS), out_ref)


def _stats3(gk, gx, q, xp, s1a, s1b, p1w, p1b, p2w, p2b, s2a, s2b, w1w, w1b):
    return _pcall(
        _stats3_body,
        grid=(GRID,),
        in_specs=[
            pl.BlockSpec((E_BLK, C), lambda i: (i, 0)),
            pl.BlockSpec((E_BLK, 16), lambda i: (i, 0)),
            pl.BlockSpec((P_BLK, C), lambda i: (i, 0)),
            pl.BlockSpec((P_BLK, 16), lambda i: (i, 0)),
            pl.BlockSpec((1, 16), lambda i: (0, 0)),
            pl.BlockSpec((1, 16), lambda i: (0, 0)),
            pl.BlockSpec((16, 16), lambda i: (0, 0)),
            pl.BlockSpec((1, 16), lambda i: (0, 0)),
            pl.BlockSpec((16, C), lambda i: (0, 0)),
            pl.BlockSpec((1, C), lambda i: (0, 0)),
            pl.BlockSpec((1, C), lambda i: (0, 0)),
            pl.BlockSpec((1, C), lambda i: (0, 0)),
            pl.BlockSpec((C, CS), lambda i: (0, 0)),
            pl.BlockSpec((1, CS), lambda i: (0, 0)),
        ],
        out_specs=[
            pl.BlockSpec((E_BLK, CS), lambda i: (i, 0)),
            pl.BlockSpec((2, 8, CS), lambda i: (0, 0, 0)),
        ],
        out_shape=[
            jax.ShapeDtypeStruct((E, CS), jnp.float32),
            jax.ShapeDtypeStruct((2, 8, CS), jnp.float32),
        ],
        scratch_shapes=[pltpu.VMEM((2, 8, CS), jnp.float32)],
    )(gk, gx, q, xp, s1a, s1b, p1w, p1b, p2w, p2b, s2a, s2b, w1w, w1b)


def _final_body(h_ref, gv_ref, gx_ref, xp_ref, f_ref, s1a_ref, s1b_ref,
                p1w_ref, p1b_ref, p2w_ref, p2b_ref, s3a_ref, s3b_ref,
                w2t_ref, w2bt_ref, out_ref):
    d = _edge_d(gx_ref[...], xp_ref[...], p1w_ref[...], p1b_ref[...],
                s1a_ref[...], s1b_ref[...], p2w_ref[...], p2b_ref[...])
    hn = jnp.maximum(h_ref[...] * s3a_ref[...] + s3b_ref[...], 0.0)
    gt = jnp.dot(hn, w2t_ref[...]) + w2bt_ref[...]
    m = gt.reshape(P_BLK, K // 8, 8, C).max(axis=1).max(axis=1)
    ex = jnp.exp(gt.reshape(P_BLK, K, C)
                 - m[:, None, :]).reshape(E_BLK, C)
    val = (gv_ref[...] + d) * ex
    num = val.reshape(P_BLK, K // 8, 8, C).sum(axis=1).sum(axis=1)
    den = ex.reshape(P_BLK, K // 8, 8, C).sum(axis=1).sum(axis=1)
    o = num / den + f_ref[...]
    out_ref[...] = jnp.where(o > 0, o, 0.1 * o)


def _final(h, gv, gx, xp, feats, s1a, s1b, p1w, p1b, p2w, p2b,
           s3a, s3b, w2t, w2bt):
    return _pcall(
        _final_body,
        grid=(GRID,),
        in_specs=[
            pl.BlockSpec((E_BLK, CS), lambda i: (i, 0)),
            pl.BlockSpec((E_BLK, C), lambda i: (i, 0)),
            pl.BlockSpec((E_BLK, 16), lambda i: (i, 0)),
            pl.BlockSpec((P_BLK, 16), lambda i: (i, 0)),
            pl.BlockSpec((P_BLK, C), lambda i: (i, 0)),
            pl.BlockSpec((1, 16), lambda i: (0, 0)),
            pl.BlockSpec((1, 16), lambda i: (0, 0)),
            pl.BlockSpec((16, 16), lambda i: (0, 0)),
            pl.BlockSpec((1, 16), lambda i: (0, 0)),
            pl.BlockSpec((16, C), lambda i: (0, 0)),
            pl.BlockSpec((1, C), lambda i: (0, 0)),
            pl.BlockSpec((1, CS), lambda i: (0, 0)),
            pl.BlockSpec((1, CS), lambda i: (0, 0)),
            pl.BlockSpec((CS, C), lambda i: (0, 0)),
            pl.BlockSpec((1, C), lambda i: (0, 0)),
        ],
        out_specs=pl.BlockSpec((P_BLK, C), lambda i: (i, 0)),
        out_shape=jax.ShapeDtypeStruct((N, C), jnp.float32),
    )(h, gv, gx, xp, feats, s1a, s1b, p1w, p1b, p2w, p2b,
      s3a, s3b, w2t, w2bt)


def _bn_fold(sums, gamma, beta):
    s = sums.sum(axis=1)
    m = s[0] / E
    v = s[1] / E - m * m
    a = gamma / jnp.sqrt(v + EPS)
    return (a.reshape(1, -1), (beta - m * a).reshape(1, -1))


def kernel(xyz, feats, nei_ind, Wq, bq, Wk, bk, Wv, bv, p1_W, p1_b,
           p1_gamma, p1_beta, p2_W, p2_b, bnw_gamma, bnw_beta, w1_W, w1_b,
           w1_gamma, w1_beta, w2_W, w2_b):
    f2 = feats[0]
    xp = jnp.pad(xyz[0], ((0, 0), (0, 13)))
    idx = nei_ind[0].reshape(E).astype(jnp.int32)

    p1w = jnp.zeros((16, 16), jnp.float32).at[:3, :3].set(p1_W)
    p1b = jnp.pad(p1_b, (0, 13)).reshape(1, 16)
    p1g = jnp.pad(p1_gamma, (0, 13))
    p1be = jnp.pad(p1_beta, (0, 13))
    p2w = jnp.zeros((16, C), jnp.float32).at[:3, :].set(p2_W)
    p2b = p2_b.reshape(1, C)

    gx = _sc_gather_one(xp, idx, 16)
    q, kf, vf = _proj(f2, Wq, bq, Wk, bk, Wv, bv)
    gk = _sc_gather_one(kf, idx, C, nsplit=2)
    gv = _sc_gather_one(vf, idx, C, nsplit=2)

    s1 = _stats1(gx, xp, p1w, p1b)
    s1a, s1b = _bn_fold(s1, p1g, p1be)

    s2 = _stats2(gk, gx, q, xp, s1a, s1b, p1w, p1b, p2w, p2b)
    s2a, s2b = _bn_fold(s2, bnw_gamma, bnw_beta)

    h, s3 = _stats3(gk, gx, q, xp, s1a, s1b, p1w, p1b, p2w, p2b,
                    s2a, s2b, w1_W, w1_b.reshape(1, CS))
    s3a, s3b = _bn_fold(s3, w1_gamma, w1_beta)

    out = _final(h, gv, gx, xp, f2, s1a, s1b, p1w, p1b, p2w, p2b,
                 s3a, s3b, jnp.tile(w2_W, (1, S)),
                 jnp.tile(w2_b, S).reshape(1, C))
    return out.reshape(1, N, C)


# speedup vs baseline: 1.1008x; 1.0268x over previous
import functools

import jax
import jax.numpy as jnp
from jax import lax
from jax.experimental import pallas as pl
from jax.experimental.pallas import tpu as pltpu
from jax.experimental.pallas import tpu_sc as plsc

N = 10000
K = 32
C = 128
MID = 128
S = 8
CS = C // S
EPS = 1e-5
E = N * K

P_BLK = 400
E_BLK = P_BLK * K
GRID = N // P_BLK
N_BLK = 2000

_pcall = pl.pallas_call


def _proj_body(f_ref, wq_ref, bq_ref, wk_ref, bk_ref, wv_ref, bv_ref,
               q_ref, k_ref, v_ref):
    f = f_ref[...]
    q_ref[...] = jnp.dot(f, wq_ref[...]) + bq_ref[...]
    k_ref[...] = jnp.dot(f, wk_ref[...]) + bk_ref[...]
    v_ref[...] = jnp.dot(f, wv_ref[...]) + bv_ref[...]


def _proj(feats, Wq, bq, Wk, bk, Wv, bv):
    row = pl.BlockSpec((N_BLK, C), lambda i: (i, 0))
    full = pl.BlockSpec((C, C), lambda i: (0, 0))
    bias = pl.BlockSpec((1, C), lambda i: (0, 0))
    return _pcall(
        _proj_body,
        grid=(N // N_BLK,),
        in_specs=[row, full, bias, full, bias, full, bias],
        out_specs=[row, row, row],
        out_shape=[jax.ShapeDtypeStruct((N, C), jnp.float32)] * 3,
    )(feats, Wq, bq.reshape(1, C), Wk, bk.reshape(1, C),
      Wv, bv.reshape(1, C))


def _sc_gather_one(tbl, idx, width, nsplit=1):
    info = plsc.get_sparse_core_info()
    nc, ns = info.num_cores, info.num_subcores
    nw = nc * ns
    b_per_w = E // nw
    ch = 80
    nb = 5
    chs = ch // nsplit
    n_it = b_per_w // (ch * nb)
    mesh = plsc.VectorSubcoreMesh(core_axis_name="c", subcore_axis_name="s")

    @functools.partial(
        pl.kernel, mesh=mesh,
        compiler_params=pltpu.CompilerParams(use_tc_tiling_on_sc=False),
        out_type=jax.ShapeDtypeStruct((E, width), jnp.float32),
        scratch_types=[
            pltpu.VMEM((b_per_w,), jnp.int32),
            pltpu.VMEM((nb, ch, width), jnp.float32),
        ] + [pltpu.SemaphoreType.DMA] * (2 * nb),
    )
    def gather(tbl_hbm, idx_hbm, out_hbm, idx_v, buf_v, *sems):
        gsem, wsem = sems[:nb], sems[nb:]
        wid = lax.axis_index("s") * nc + lax.axis_index("c")
        base = wid * b_per_w
        pltpu.sync_copy(idx_hbm.at[pl.ds(base, b_per_w)], idx_v)

        def drain_write(b):
            pltpu.make_async_copy(buf_v.at[b], out_hbm.at[pl.ds(base, ch)],
                                  wsem[b]).wait()

        def body(it, carry):
            descs = []
            for b in range(nb):
                c = it * nb + b

                @pl.when(it > 0)
                def _(b=b):
                    drain_write(b)

                descs.append([
                    pltpu.async_copy(
                        tbl_hbm.at[idx_v.at[pl.ds(c * ch + s * chs, chs)]],
                        buf_v.at[b, pl.ds(s * chs, chs)], gsem[b])
                    for s in range(nsplit)])
            for b in range(nb):
                off = base + (it * nb + b) * ch
                for dsc in descs[b]:
                    dsc.wait()
                pltpu.async_copy(buf_v.at[b], out_hbm.at[pl.ds(off, ch)],
                                 wsem[b])
            return carry

        lax.fori_loop(0, n_it, body, 0)
        for b in range(nb):
            drain_write(b)

    return gather(tbl, idx)


def _edge_d(gx, xp, p1w, p1b, s1a, s1b, p2w, p2b):
    gx3 = gx.reshape(P_BLK, K, 16)
    dx = (gx3 - xp[:, None, :]).reshape(E_BLK, 16)
    x = jnp.dot(dx, p1w) + p1b
    r = jnp.maximum(x * s1a + s1b, 0.0)
    return jnp.dot(r, p2w) + p2b


def _acc_update(acc, i, vals, out_ref):
    @pl.when(i == 0)
    def _():
        acc[...] = jnp.zeros_like(acc[...])

    acc[0] += vals.sum(axis=0)
    acc[1] += (vals * vals).sum(axis=0)

    @pl.when(i == pl.num_programs(0) - 1)
    def _():
        out_ref[...] = acc[...]


def _stats1_body(gx_ref, xp_ref, p1w_ref, p1b_ref, out_ref, acc):
    i = pl.program_id(0)
    gx3 = gx_ref[...].reshape(P_BLK, K, 16)
    dx = (gx3 - xp_ref[...][:, None, :]).reshape(E_BLK, 16)
    x = jnp.dot(dx, p1w_ref[...]) + p1b_ref[...]
    _acc_update(acc, i, x.reshape(E_BLK // 8, 8, 16), out_ref)


def _stats1(gx, xp, p1w, p1b):
    return _pcall(
        _stats1_body,
        grid=(GRID,),
        in_specs=[
            pl.BlockSpec((E_BLK, 16), lambda i: (i, 0)),
            pl.BlockSpec((P_BLK, 16), lambda i: (i, 0)),
            pl.BlockSpec((16, 16), lambda i: (0, 0)),
            pl.BlockSpec((1, 16), lambda i: (0, 0)),
        ],
        out_specs=pl.BlockSpec((2, 8, 16), lambda i: (0, 0, 0)),
        out_shape=jax.ShapeDtypeStruct((2, 8, 16), jnp.float32),
        scratch_shapes=[pltpu.VMEM((2, 8, 16), jnp.float32)],
    )(gx, xp, p1w, p1b)


def _stats2_body(gk_ref, gx_ref, q_ref, xp_ref, s1a_ref, s1b_ref,
                 p1w_ref, p1b_ref, p2w_ref, p2b_ref, out_ref, acc):
    i = pl.program_id(0)
    d = _edge_d(gx_ref[...], xp_ref[...], p1w_ref[...], p1b_ref[...],
                s1a_ref[...], s1b_ref[...], p2w_ref[...], p2b_ref[...])
    w3 = (gk_ref[...].reshape(P_BLK, K, C)
          - q_ref[...][:, None, :] + d.reshape(P_BLK, K, C))
    _acc_update(acc, i, w3.reshape(E_BLK // 8, 8, C), out_ref)


def _stats2(gk, gx, q, xp, s1a, s1b, p1w, p1b, p2w, p2b):
    return _pcall(
        _stats2_body,
        grid=(GRID,),
        in_specs=[
            pl.BlockSpec((E_BLK, C), lambda i: (i, 0)),
            pl.BlockSpec((E_BLK, 16), lambda i: (i, 0)),
            pl.BlockSpec((P_BLK, C), lambda i: (i, 0)),
            pl.BlockSpec((P_BLK, 16), lambda i: (i, 0)),
            pl.BlockSpec((1, 16), lambda i: (0, 0)),
            pl.BlockSpec((1, 16), lambda i: (0, 0)),
            pl.BlockSpec((16, 16), lambda i: (0, 0)),
            pl.BlockSpec((1, 16), lambda i: (0, 0)),
            pl.BlockSpec((16, C), lambda i: (0, 0)),
            pl.BlockSpec((1, C), lambda i: (0, 0)),
        ],
        out_specs=pl.BlockSpec((2, 8, C), lambda i: (0, 0, 0)),
        out_shape=jax.ShapeDtypeStruct((2, 8, C), jnp.float32),
        scratch_shapes=[pltpu.VMEM((2, 8, C), jnp.float32)],
    )(gk, gx, q, xp, s1a, s1b, p1w, p1b, p2w, p2b)


def _stats3_body(gk_ref, gx_ref, q_ref, xp_ref, s1a_ref, s1b_ref,
                 p1w_ref, p1b_ref, p2w_ref, p2b_ref, s2a_ref, s2b_ref,
                 w1w_ref, w1b_ref, h_ref, out_ref, acc):
    i = pl.program_id(0)
    d = _edge_d(gx_ref[...], xp_ref[...], p1w_ref[...], p1b_ref[...],
                s1a_ref[...], s1b_ref[...], p2w_ref[...], p2b_ref[...])
    w3 = (gk_ref[...].reshape(P_BLK, K, C)
          - q_ref[...][:, None, :] + d.reshape(P_BLK, K, C))
    wn = jnp.maximum(w3.reshape(E_BLK, C) * s2a_ref[...] + s2b_ref[...], 0.0)
    h = jnp.dot(wn, w1w_ref[...]) + w1b_ref[...]
    h_ref[...] = h
    _acc_update(acc, i, h.reshape(E_BLK // 8, 8, CS), out_ref)


def _stats3(gk, gx, q, xp, s1a, s1b, p1w, p1b, p2w, p2b, s2a, s2b, w1w, w1b):
    return _pcall(
        _stats3_body,
        grid=(GRID,),
        in_specs=[
            pl.BlockSpec((E_BLK, C), lambda i: (i, 0)),
            pl.BlockSpec((E_BLK, 16), lambda i: (i, 0)),
            pl.BlockSpec((P_BLK, C), lambda i: (i, 0)),
            pl.BlockSpec((P_BLK, 16), lambda i: (i, 0)),
            pl.BlockSpec((1, 16), lambda i: (0, 0)),
            pl.BlockSpec((1, 16), lambda i: (0, 0)),
            pl.BlockSpec((16, 16), lambda i: (0, 0)),
            pl.BlockSpec((1, 16), lambda i: (0, 0)),
            pl.BlockSpec((16, C), lambda i: (0, 0)),
            pl.BlockSpec((1, C), lambda i: (0, 0)),
            pl.BlockSpec((1, C), lambda i: (0, 0)),
            pl.BlockSpec((1, C), lambda i: (0, 0)),
            pl.BlockSpec((C, CS), lambda i: (0, 0)),
            pl.BlockSpec((1, CS), lambda i: (0, 0)),
        ],
        out_specs=[
            pl.BlockSpec((E_BLK, CS), lambda i: (i, 0)),
            pl.BlockSpec((2, 8, CS), lambda i: (0, 0, 0)),
        ],
        out_shape=[
            jax.ShapeDtypeStruct((E, CS), jnp.float32),
            jax.ShapeDtypeStruct((2, 8, CS), jnp.float32),
        ],
        scratch_shapes=[pltpu.VMEM((2, 8, CS), jnp.float32)],
    )(gk, gx, q, xp, s1a, s1b, p1w, p1b, p2w, p2b, s2a, s2b, w1w, w1b)


def _final_body(h_ref, gv_ref, gx_ref, xp_ref, f_ref, s1a_ref, s1b_ref,
                p1w_ref, p1b_ref, p2w_ref, p2b_ref, s3a_ref, s3b_ref,
                w2t_ref, w2bt_ref, out_ref):
    d = _edge_d(gx_ref[...], xp_ref[...], p1w_ref[...], p1b_ref[...],
                s1a_ref[...], s1b_ref[...], p2w_ref[...], p2b_ref[...])
    hn = jnp.maximum(h_ref[...] * s3a_ref[...] + s3b_ref[...], 0.0)
    gt = jnp.dot(hn, w2t_ref[...]) + w2bt_ref[...]
    m = gt.reshape(P_BLK, K // 8, 8, C).max(axis=1).max(axis=1)
    ex = jnp.exp(gt.reshape(P_BLK, K, C)
                 - m[:, None, :]).reshape(E_BLK, C)
    val = (gv_ref[...] + d) * ex
    num = val.reshape(P_BLK, K // 8, 8, C).sum(axis=1).sum(axis=1)
    den = ex.reshape(P_BLK, K // 8, 8, C).sum(axis=1).sum(axis=1)
    o = num / den + f_ref[...]
    out_ref[...] = jnp.where(o > 0, o, 0.1 * o)


def _final(h, gv, gx, xp, feats, s1a, s1b, p1w, p1b, p2w, p2b,
           s3a, s3b, w2t, w2bt):
    return _pcall(
        _final_body,
        grid=(GRID,),
        in_specs=[
            pl.BlockSpec((E_BLK, CS), lambda i: (i, 0)),
            pl.BlockSpec((E_BLK, C), lambda i: (i, 0)),
            pl.BlockSpec((E_BLK, 16), lambda i: (i, 0)),
            pl.BlockSpec((P_BLK, 16), lambda i: (i, 0)),
            pl.BlockSpec((P_BLK, C), lambda i: (i, 0)),
            pl.BlockSpec((1, 16), lambda i: (0, 0)),
            pl.BlockSpec((1, 16), lambda i: (0, 0)),
            pl.BlockSpec((16, 16), lambda i: (0, 0)),
            pl.BlockSpec((1, 16), lambda i: (0, 0)),
            pl.BlockSpec((16, C), lambda i: (0, 0)),
            pl.BlockSpec((1, C), lambda i: (0, 0)),
            pl.BlockSpec((1, CS), lambda i: (0, 0)),
            pl.BlockSpec((1, CS), lambda i: (0, 0)),
            pl.BlockSpec((CS, C), lambda i: (0, 0)),
            pl.BlockSpec((1, C), lambda i: (0, 0)),
        ],
        out_specs=pl.BlockSpec((P_BLK, C), lambda i: (i, 0)),
        out_shape=jax.ShapeDtypeStruct((N, C), jnp.float32),
    )(h, gv, gx, xp, feats, s1a, s1b, p1w, p1b, p2w, p2b,
      s3a, s3b, w2t, w2bt)


def _bn_fold(sums, gamma, beta):
    s = sums.sum(axis=1)
    m = s[0] / E
    v = s[1] / E - m * m
    a = gamma / jnp.sqrt(v + EPS)
    return (a.reshape(1, -1), (beta - m * a).reshape(1, -1))


def kernel(xyz, feats, nei_ind, Wq, bq, Wk, bk, Wv, bv, p1_W, p1_b,
           p1_gamma, p1_beta, p2_W, p2_b, bnw_gamma, bnw_beta, w1_W, w1_b,
           w1_gamma, w1_beta, w2_W, w2_b):
    f2 = feats[0]
    xp = jnp.pad(xyz[0], ((0, 0), (0, 13)))
    idx = nei_ind[0].reshape(E).astype(jnp.int32)

    p1w = jnp.zeros((16, 16), jnp.float32).at[:3, :3].set(p1_W)
    p1b = jnp.pad(p1_b, (0, 13)).reshape(1, 16)
    p1g = jnp.pad(p1_gamma, (0, 13))
    p1be = jnp.pad(p1_beta, (0, 13))
    p2w = jnp.zeros((16, C), jnp.float32).at[:3, :].set(p2_W)
    p2b = p2_b.reshape(1, C)

    gx = _sc_gather_one(xp, idx, 16)
    q, kf, vf = _proj(f2, Wq, bq, Wk, bk, Wv, bv)
    gk = _sc_gather_one(kf, idx, C, nsplit=2)
    gv = _sc_gather_one(vf, idx, C, nsplit=2)

    s1 = _stats1(gx, xp, p1w, p1b)
    s1a, s1b = _bn_fold(s1, p1g, p1be)

    s2 = _stats2(gk, gx, q, xp, s1a, s1b, p1w, p1b, p2w, p2b)
    s2a, s2b = _bn_fold(s2, bnw_gamma, bnw_beta)

    h, s3 = _stats3(gk, gx, q, xp, s1a, s1b, p1w, p1b, p2w, p2b,
                    s2a, s2b, w1_W, w1_b.reshape(1, CS))
    s3a, s3b = _bn_fold(s3, w1_gamma, w1_beta)

    out = _final(h, gv, gx, xp, f2, s1a, s1b, p1w, p1b, p2w, p2b,
                 s3a, s3b, jnp.tile(w2_W, (1, S)),
                 jnp.tile(w2_b, S).reshape(1, C))
    return out.reshape(1, N, C)
